# baseline (speedup 1.0000x reference)
import functools

import jax
import jax.numpy as jnp
from jax import lax
from jax.experimental import pallas as pl
from jax.experimental.pallas import tpu as pltpu

_NUM_CLASSES = 80
_REG_MAX = 16
_NB = _REG_MAX + 1
_LANES = 128
_W = 4096
_RB = _W // _LANES


def _eye(n):
    i0 = lax.broadcasted_iota(jnp.int32, (n, n), 0)
    i1 = lax.broadcasted_iota(jnp.int32, (n, n), 1)
    return (i0 == i1).astype(jnp.float32)


def _lanes_t(lhs, blk, k):
    out = lax.dot_general(lhs, blk, (((1,), (1,)), ((), ())),
                          preferred_element_type=jnp.float32)
    return out[:k].reshape(k, _RB, _LANES)


def _loss_body(sc_ref, pd_ref, pb_ref, ab_ref, ap_ref, lab_ref, ssum_ref,
               l1_ref, iou_ref, dfl_ref, acc_ref):
    step = pl.program_id(0)
    nsteps = pl.num_programs(0)

    @pl.when(step == 0)
    def _init():
        for i in range(5):
            acc_ref[i] = 0.0

    mask = lab_ref[...] != _NUM_CLASSES
    maskf = mask.astype(jnp.float32)

    ones8 = jnp.ones((8, _NUM_CLASSES), jnp.float32)
    rs = _lanes_t(ones8, sc_ref[0], 1)[0]
    eye2 = _eye(2)
    pbt = _lanes_t(eye2, pb_ref[0], 2)
    abt = _lanes_t(eye2, ab_ref[0], 2)
    apt = _lanes_t(jnp.ones((1, 1), jnp.float32), ap_ref[0], 1)
    pdt = _lanes_t(_eye(2 * _NB), pd_ref[0], 2 * _NB)

    x0p = pbt[0]
    x1p = pbt[1]
    x0a = abt[0]
    x1a = abt[1]
    ap = apt[0]

    l1sum = jnp.sum((jnp.abs(x0p - x0a) + jnp.abs(x1p - x1a)) * maskf)

    inter = jnp.maximum(jnp.minimum(x1p, x1a) - jnp.maximum(x0p, x0a), 0.0)
    union = (x1p - x0p) + (x1a - x0a) - inter
    union_safe = jnp.where(mask, union, 1.0)
    tiou = jnp.where(mask, inter / union_safe, 0.0)
    iousum = jnp.sum(jnp.where(mask, 1.0 - tiou, 0.0))
    npos = jnp.sum(maskf)

    bw = rs * maskf
    bwsum = jnp.sum(bw)

    ltrb_l = jnp.clip(ap - x0a, 0.0, _REG_MAX - 0.01)
    ltrb_r = jnp.clip(x1a - ap, 0.0, _REG_MAX - 0.01)
    iota3 = lax.broadcasted_iota(jnp.int32, (_NB, _RB, _LANES), 0)

    def _dfl_half(x, ltrb):
        logS = jnp.log(jnp.sum(jnp.exp(x), axis=0))
        t = ltrb.astype(jnp.int32)
        xt = jnp.sum(jnp.where(iota3 == t[None], x, 0.0), axis=0)
        xt1 = jnp.sum(jnp.where(iota3 == t[None] + 1, x, 0.0), axis=0)
        wl = (t + 1).astype(jnp.float32) - ltrb
        wr = 1.0 - wl
        return (logS - xt) * wl + (logS - xt1) * wr

    dfl = 0.5 * (_dfl_half(pdt[:_NB], ltrb_l) + _dfl_half(pdt[_NB:], ltrb_r))
    dflsum = jnp.sum(dfl * bw)

    acc_ref[0] += npos
    acc_ref[1] += l1sum
    acc_ref[2] += iousum
    acc_ref[3] += bwsum
    acc_ref[4] += dflsum

    @pl.when(step == nsteps - 1)
    def _finish():
        np_ = acc_ref[0]
        ssum = ssum_ref[0]
        l1_ref[0] = acc_ref[1] / (np_ * 2.0)
        iou_ref[0] = (acc_ref[2] / np_) * acc_ref[3] / ssum
        dfl_ref[0] = acc_ref[4] / ssum


@functools.partial(jax.jit, static_argnames=("interpret",))
def _run(pred_dist, pred_bboxes, anchor_points, assigned_labels,
         assigned_bboxes, assigned_scores, assigned_scores_sum,
         interpret=False):
    B, L = assigned_labels.shape
    N = B * L
    NR = N // _LANES

    S = L // _W
    lab = assigned_labels.reshape(NR, _LANES)
    ssum = assigned_scores_sum.reshape(1)

    out = pl.pallas_call(
        _loss_body,
        grid=(N // _W,),
        in_specs=[
            pl.BlockSpec((1, _W, _NUM_CLASSES), lambda i: (i // S, i % S, 0)),
            pl.BlockSpec((1, _W, 2 * _NB), lambda i: (i // S, i % S, 0)),
            pl.BlockSpec((1, _W, 2), lambda i: (i // S, i % S, 0)),
            pl.BlockSpec((1, _W, 2), lambda i: (i // S, i % S, 0)),
            pl.BlockSpec((1, _W, 1), lambda i: (i // S, i % S, 0)),
            pl.BlockSpec((_RB, _LANES), lambda i: (i, 0)),
            pl.BlockSpec(memory_space=pltpu.SMEM),
        ],
        out_specs=[
            pl.BlockSpec(memory_space=pltpu.SMEM),
            pl.BlockSpec(memory_space=pltpu.SMEM),
            pl.BlockSpec(memory_space=pltpu.SMEM),
        ],
        out_shape=[
            jax.ShapeDtypeStruct((1,), jnp.float32),
            jax.ShapeDtypeStruct((1,), jnp.float32),
            jax.ShapeDtypeStruct((1,), jnp.float32),
        ],
        scratch_shapes=[pltpu.SMEM((8,), jnp.float32)],
        compiler_params=pltpu.CompilerParams(
            dimension_semantics=("arbitrary",)),
        interpret=interpret,
    )(assigned_scores, pred_dist, pred_bboxes, assigned_bboxes,
      anchor_points, lab, ssum)
    return (out[0][0], out[1][0], out[2][0])


def kernel(pred_dist, pred_bboxes, anchor_points, assigned_labels,
           assigned_bboxes, assigned_scores, assigned_scores_sum):
    return _run(pred_dist, pred_bboxes, anchor_points, assigned_labels,
                assigned_bboxes, assigned_scores, assigned_scores_sum)


# speedup vs baseline: 2.8495x; 2.8495x over previous
import functools

import jax
import jax.numpy as jnp
from jax import lax
from jax.experimental import pallas as pl
from jax.experimental.pallas import tpu as pltpu

_NUM_CLASSES = 80
_REG_MAX = 16
_NB = _REG_MAX + 1
_LANES = 128
_W = 8192
_RB = _W // _LANES


def _fmt_body(sc_ref, rs_ref):
    ones8 = jnp.ones((8, _NUM_CLASSES), jnp.float32)
    rs8 = lax.dot_general(ones8, sc_ref[...], (((1,), (1,)), ((), ())),
                          preferred_element_type=jnp.float32)
    rs_ref[...] = rs8[0:1].reshape(1, _RB, _LANES)


def _loss_body(pdt_ref, rs_ref, pbt_ref, apt_ref, lab_ref, abt_ref, ssum_ref,
               l1_ref, iou_ref, dfl_ref, acc_ref):
    step = pl.program_id(0)
    nsteps = pl.num_programs(0)

    @pl.when(step == 0)
    def _init():
        for i in range(5):
            acc_ref[i] = 0.0

    mask = lab_ref[...] != _NUM_CLASSES
    maskf = mask.astype(jnp.float32)

    x0p = pbt_ref[0]
    x1p = pbt_ref[1]
    x0a = abt_ref[0]
    x1a = abt_ref[1]
    ap = apt_ref[0]

    l1sum = jnp.sum((jnp.abs(x0p - x0a) + jnp.abs(x1p - x1a)) * maskf)

    inter = jnp.maximum(jnp.minimum(x1p, x1a) - jnp.maximum(x0p, x0a), 0.0)
    union = (x1p - x0p) + (x1a - x0a) - inter
    union_safe = jnp.where(mask, union, 1.0)
    tiou = jnp.where(mask, inter / union_safe, 0.0)
    iousum = jnp.sum(jnp.where(mask, 1.0 - tiou, 0.0))
    npos = jnp.sum(maskf)

    bw = rs_ref[0] * maskf
    bwsum = jnp.sum(bw)

    ltrb_l = jnp.clip(ap - x0a, 0.0, _REG_MAX - 0.01)
    ltrb_r = jnp.clip(x1a - ap, 0.0, _REG_MAX - 0.01)
    pd = pdt_ref[...]
    iota3 = lax.broadcasted_iota(jnp.int32, (_NB, _RB, _LANES), 0)

    def _dfl_half(x, ltrb):
        logS = jnp.log(jnp.sum(jnp.exp(x), axis=0))
        t = ltrb.astype(jnp.int32)
        xt = jnp.sum(jnp.where(iota3 == t[None], x, 0.0), axis=0)
        xt1 = jnp.sum(jnp.where(iota3 == t[None] + 1, x, 0.0), axis=0)
        wl = (t + 1).astype(jnp.float32) - ltrb
        wr = 1.0 - wl
        return (logS - xt) * wl + (logS - xt1) * wr

    dfl = 0.5 * (_dfl_half(pd[:_NB], ltrb_l) + _dfl_half(pd[_NB:], ltrb_r))
    dflsum = jnp.sum(dfl * bw)

    acc_ref[0] += npos
    acc_ref[1] += l1sum
    acc_ref[2] += iousum
    acc_ref[3] += bwsum
    acc_ref[4] += dflsum

    @pl.when(step == nsteps - 1)
    def _finish():
        np_ = acc_ref[0]
        ssum = ssum_ref[0]
        l1_ref[0] = acc_ref[1] / (np_ * 2.0)
        iou_ref[0] = (acc_ref[2] / np_) * acc_ref[3] / ssum
        dfl_ref[0] = acc_ref[4] / ssum


@functools.partial(jax.jit, static_argnames=("interpret",))
def _run(pred_dist, pred_bboxes, anchor_points, assigned_labels,
         assigned_bboxes, assigned_scores, assigned_scores_sum,
         interpret=False):
    B, L = assigned_labels.shape
    N = B * L
    NR = N // _LANES
    R = N // _W

    sc = assigned_scores.reshape(N, _NUM_CLASSES)
    rsT = pl.pallas_call(
        _fmt_body,
        grid=(R,),
        in_specs=[pl.BlockSpec((_W, _NUM_CLASSES), lambda i: (i, 0))],
        out_specs=pl.BlockSpec((1, _RB, _LANES), lambda i: (0, i, 0)),
        out_shape=jax.ShapeDtypeStruct((1, NR, _LANES), jnp.float32),
        compiler_params=pltpu.CompilerParams(
            dimension_semantics=("arbitrary",)),
        interpret=interpret,
    )(sc)

    pdT = jnp.transpose(pred_dist.reshape(NR, _LANES, 2 * _NB), (2, 0, 1))
    pbT = jnp.transpose(pred_bboxes.reshape(NR, _LANES, 2), (2, 0, 1))
    abT = jnp.transpose(assigned_bboxes.reshape(NR, _LANES, 2), (2, 0, 1))
    apT = jnp.transpose(anchor_points.reshape(NR, _LANES, 1), (2, 0, 1))
    lab = assigned_labels.reshape(NR, _LANES)
    ssum = assigned_scores_sum.reshape(1)

    row_spec = pl.BlockSpec((_RB, _LANES), lambda i: (i, 0))
    out = pl.pallas_call(
        _loss_body,
        grid=(R,),
        in_specs=[
            pl.BlockSpec((2 * _NB, _RB, _LANES), lambda i: (0, i, 0)),
            pl.BlockSpec((1, _RB, _LANES), lambda i: (0, i, 0)),
            pl.BlockSpec((2, _RB, _LANES), lambda i: (0, i, 0)),
            pl.BlockSpec((1, _RB, _LANES), lambda i: (0, i, 0)),
            row_spec,
            pl.BlockSpec((2, _RB, _LANES), lambda i: (0, i, 0)),
            pl.BlockSpec(memory_space=pltpu.SMEM),
        ],
        out_specs=[
            pl.BlockSpec(memory_space=pltpu.SMEM),
            pl.BlockSpec(memory_space=pltpu.SMEM),
            pl.BlockSpec(memory_space=pltpu.SMEM),
        ],
        out_shape=[
            jax.ShapeDtypeStruct((1,), jnp.float32),
            jax.ShapeDtypeStruct((1,), jnp.float32),
            jax.ShapeDtypeStruct((1,), jnp.float32),
        ],
        scratch_shapes=[pltpu.SMEM((8,), jnp.float32)],
        compiler_params=pltpu.CompilerParams(
            dimension_semantics=("arbitrary",)),
        interpret=interpret,
    )(pdT, rsT, pbT, apT, lab, abT, ssum)
    return (out[0][0], out[1][0], out[2][0])


def kernel(pred_dist, pred_bboxes, anchor_points, assigned_labels,
           assigned_bboxes, assigned_scores, assigned_scores_sum):
    return _run(pred_dist, pred_bboxes, anchor_points, assigned_labels,
                assigned_bboxes, assigned_scores, assigned_scores_sum)
